# baseline (device time: 275034 ns/iter reference)
import jax
import jax.numpy as jnp
from jax import lax
from jax.experimental import pallas as pl
from jax.experimental.pallas import tpu as pltpu

N_DEV = 32
DH = 64
WINDOW = 128


def _all_reduce(partial):
    m, n = partial.shape
    assert m % N_DEV == 0
    bf16 = jnp.bfloat16

    rs_sizes = [m >> (k + 1) for k in range(5)]
    rs_offs = [sum(rs_sizes[:k]) for k in range(5)]
    rs_total = sum(rs_sizes)

    def body(x_ref, out_ref, acc_ref, send_stage, rs_recv, ag_buf,
             send_sems, recv_sems):
        i = lax.axis_index("i")
        z = i // 8
        p = lax.rem(i, 8)
        y = p // 2
        x = jnp.bitwise_xor(lax.rem(p, 2), lax.rem(y, 2))

        def ring_idx(xx, yy, zz):
            pp = 2 * yy + jnp.bitwise_xor(xx, lax.rem(yy, 2))
            return 8 * zz + pp

        partners = [
            ring_idx(jnp.bitwise_xor(x, 1), y, z),
            ring_idx(x, jnp.bitwise_xor(y, 1), z),
            ring_idx(x, y, jnp.bitwise_xor(z, 1)),
            ring_idx(x, jnp.bitwise_xor(y, 2), z),
            ring_idx(x, y, jnp.bitwise_xor(z, 2)),
        ]
        bits = [
            lax.rem(x, 2),
            lax.rem(y, 2),
            lax.rem(z, 2),
            lax.rem(y // 2, 2),
            lax.rem(z // 2, 2),
        ]

        barrier_sem = pltpu.get_barrier_semaphore()
        for pk in partners:
            pl.semaphore_signal(
                barrier_sem, inc=1,
                device_id=(pk,), device_id_type=pl.DeviceIdType.MESH,
            )
        pl.semaphore_wait(barrier_sem, len(partners))

        acc_ref[:, :] = x_ref[:, :]

        seg_start = i * 0
        for k in range(5):
            half = rs_sizes[k]
            b = bits[k]
            send_off = seg_start + (1 - b) * half
            keep_off = seg_start + b * half
            send_stage[pl.ds(0, half), :] = acc_ref[
                pl.ds(send_off, half), :
            ].astype(bf16)
            rdma = pltpu.make_async_remote_copy(
                src_ref=send_stage.at[pl.ds(0, half)],
                dst_ref=rs_recv.at[pl.ds(rs_offs[k], half)],
                send_sem=send_sems.at[k],
                recv_sem=recv_sems.at[k],
                device_id=(partners[k],),
                device_id_type=pl.DeviceIdType.MESH,
            )
            rdma.start()
            rdma.wait()
            acc_ref[pl.ds(keep_off, half), :] = (
                acc_ref[pl.ds(keep_off, half), :]
                + rs_recv[pl.ds(rs_offs[k], half), :].astype(jnp.float32)
            )
            seg_start = keep_off

        own = m // N_DEV
        ag_buf[pl.ds(seg_start, own), :] = acc_ref[
            pl.ds(seg_start, own), :
        ].astype(bf16)
        own_start = seg_start
        for j in range(5):
            r = 4 - j
            b = bits[r]
            rdma = pltpu.make_async_remote_copy(
                src_ref=ag_buf.at[pl.ds(own_start, own)],
                dst_ref=ag_buf.at[pl.ds(own_start, own)],
                send_sem=send_sems.at[5 + j],
                recv_sem=recv_sems.at[5 + j],
                device_id=(partners[r],),
                device_id_type=pl.DeviceIdType.MESH,
            )
            rdma.start()
            rdma.wait()
            own_start = own_start - b * own
            own = own * 2

        out_ref[:, :] = ag_buf[:, :].astype(jnp.float32)

    return pl.pallas_call(
        body,
        out_shape=jax.ShapeDtypeStruct((m, n), partial.dtype),
        in_specs=[pl.BlockSpec(memory_space=pltpu.VMEM)],
        out_specs=pl.BlockSpec(memory_space=pltpu.VMEM),
        scratch_shapes=[
            pltpu.VMEM((m, n), partial.dtype),
            pltpu.VMEM((m // 2, n), bf16),
            pltpu.VMEM((rs_total, n), bf16),
            pltpu.VMEM((m, n), bf16),
            pltpu.SemaphoreType.DMA((10,)),
            pltpu.SemaphoreType.DMA((10,)),
        ],
        compiler_params=pltpu.CompilerParams(collective_id=0),
    )(partial)


def _partial_attn(x, Wq, K, V, Wo):
    B, Sq, E = x.shape
    h_per = Wq.shape[1] // DH
    bf16 = jnp.bfloat16

    def body(x_ref, wq_ref, k_hbm, v_hbm, wo_ref, out_ref,
             k_ref, v_ref, dma_sems):
        h0 = lax.axis_index("i") * h_per
        cp_k = pltpu.make_async_copy(
            k_hbm.at[:, :, pl.ds(h0, h_per), :], k_ref, dma_sems.at[0])
        cp_v = pltpu.make_async_copy(
            v_hbm.at[:, :, pl.ds(h0, h_per), :], v_ref, dma_sems.at[1])
        cp_k.start()
        cp_v.start()

        qi = lax.broadcasted_iota(jnp.int32, (Sq, Sq), 0)
        ki = lax.broadcasted_iota(jnp.int32, (Sq, Sq), 1)
        neg = jnp.float32(-1e9)
        band = jnp.abs(qi - ki) <= WINDOW
        cp_k.wait()
        cp_v.wait()
        for b in range(B):
            xb = x_ref[b, :, :].astype(bf16)
            acc = jnp.zeros((Sq, E), jnp.float32)
            for h in range(h_per):
                wq_h = wq_ref[:, h * DH:(h + 1) * DH].astype(bf16)
                q = jnp.dot(xb, wq_h,
                            preferred_element_type=jnp.float32)
                q = (q * 0.125).astype(bf16)
                k = k_ref[b, :, h, :].astype(bf16)
                s = lax.dot_general(
                    q, k, (((1,), (1,)), ((), ())),
                    preferred_element_type=jnp.float32)
                s = jnp.where(band, s, neg)
                s = s - jnp.max(s, axis=1, keepdims=True)
                e = jnp.exp(s)
                w = e / jnp.sum(e, axis=1, keepdims=True)
                v = v_ref[b, :, h, :].astype(bf16)
                ctx = jnp.dot(w.astype(bf16), v,
                              preferred_element_type=jnp.float32)
                wo_h = wo_ref[h * DH:(h + 1) * DH, :].astype(bf16)
                acc = acc + jnp.dot(ctx.astype(bf16), wo_h,
                                    preferred_element_type=jnp.float32)
            out_ref[b * Sq:(b + 1) * Sq, :] = acc

    return pl.pallas_call(
        body,
        out_shape=jax.ShapeDtypeStruct((B * Sq, E), jnp.float32),
        in_specs=[
            pl.BlockSpec(memory_space=pltpu.VMEM),
            pl.BlockSpec(memory_space=pltpu.VMEM),
            pl.BlockSpec(memory_space=pltpu.MemorySpace.HBM),
            pl.BlockSpec(memory_space=pltpu.MemorySpace.HBM),
            pl.BlockSpec(memory_space=pltpu.VMEM),
        ],
        out_specs=pl.BlockSpec(memory_space=pltpu.VMEM),
        scratch_shapes=[
            pltpu.VMEM((B, Sq, h_per, DH), jnp.float32),
            pltpu.VMEM((B, Sq, h_per, DH), jnp.float32),
            pltpu.SemaphoreType.DMA((2,)),
        ],
    )(x, Wq, K, V, Wo)


def kernel(x, Wq, K_ext, V_ext, Wo):
    B, Sq, E = x.shape

    partial = _partial_attn(x, Wq, K_ext, V_ext, Wo)
    out = _all_reduce(partial)
    return out.reshape(B, Sq, E)


# device time: 183339 ns/iter; 1.5001x vs baseline; 1.5001x over previous
import jax
import jax.numpy as jnp
from jax import lax
from jax.experimental import pallas as pl
from jax.experimental.pallas import tpu as pltpu

N_DEV = 32
DH = 64
WINDOW = 128


def _all_reduce(partial):
    m, n = partial.shape
    assert m % N_DEV == 0
    bf16 = jnp.bfloat16

    rs_sizes = [m >> (k + 1) for k in range(5)]
    rs_offs = [sum(rs_sizes[:k]) for k in range(5)]
    rs_total = sum(rs_sizes)

    def body(x_ref, out_ref, acc_ref, send_stage, rs_recv, ag_buf,
             send_sems, recv_sems):
        i = lax.axis_index("i")
        z = i // 8
        p = lax.rem(i, 8)
        y = p // 2
        x = jnp.bitwise_xor(lax.rem(p, 2), lax.rem(y, 2))

        def ring_idx(xx, yy, zz):
            pp = 2 * yy + jnp.bitwise_xor(xx, lax.rem(yy, 2))
            return 8 * zz + pp

        partners = [
            ring_idx(jnp.bitwise_xor(x, 1), y, z),
            ring_idx(x, jnp.bitwise_xor(y, 1), z),
            ring_idx(x, y, jnp.bitwise_xor(z, 1)),
            ring_idx(x, jnp.bitwise_xor(y, 2), z),
            ring_idx(x, y, jnp.bitwise_xor(z, 2)),
        ]
        bits = [
            lax.rem(x, 2),
            lax.rem(y, 2),
            lax.rem(z, 2),
            lax.rem(y // 2, 2),
            lax.rem(z // 2, 2),
        ]

        barrier_sem = pltpu.get_barrier_semaphore()
        for pk in partners:
            pl.semaphore_signal(
                barrier_sem, inc=1,
                device_id=(pk,), device_id_type=pl.DeviceIdType.MESH,
            )
        pl.semaphore_wait(barrier_sem, len(partners))

        acc_ref[:, :] = x_ref[:, :]

        seg_start = i * 0
        for k in range(5):
            half = rs_sizes[k]
            b = bits[k]
            send_off = seg_start + (1 - b) * half
            keep_off = seg_start + b * half
            send_stage[pl.ds(0, half), :] = acc_ref[
                pl.ds(send_off, half), :
            ].astype(bf16)
            rdma = pltpu.make_async_remote_copy(
                src_ref=send_stage.at[pl.ds(0, half)],
                dst_ref=rs_recv.at[pl.ds(rs_offs[k], half)],
                send_sem=send_sems.at[k],
                recv_sem=recv_sems.at[k],
                device_id=(partners[k],),
                device_id_type=pl.DeviceIdType.MESH,
            )
            rdma.start()
            rdma.wait()
            acc_ref[pl.ds(keep_off, half), :] = (
                acc_ref[pl.ds(keep_off, half), :]
                + rs_recv[pl.ds(rs_offs[k], half), :].astype(jnp.float32)
            )
            seg_start = keep_off

        own = m // N_DEV
        ag_buf[pl.ds(seg_start, own), :] = acc_ref[
            pl.ds(seg_start, own), :
        ].astype(bf16)
        own_start = seg_start
        for j in range(5):
            r = 4 - j
            b = bits[r]
            rdma = pltpu.make_async_remote_copy(
                src_ref=ag_buf.at[pl.ds(own_start, own)],
                dst_ref=ag_buf.at[pl.ds(own_start, own)],
                send_sem=send_sems.at[5 + j],
                recv_sem=recv_sems.at[5 + j],
                device_id=(partners[r],),
                device_id_type=pl.DeviceIdType.MESH,
            )
            rdma.start()
            rdma.wait()
            own_start = own_start - b * own
            own = own * 2

        out_ref[:, :] = ag_buf[:, :].astype(jnp.float32)

    return pl.pallas_call(
        body,
        out_shape=jax.ShapeDtypeStruct((m, n), partial.dtype),
        in_specs=[pl.BlockSpec(memory_space=pltpu.VMEM)],
        out_specs=pl.BlockSpec(memory_space=pltpu.VMEM),
        scratch_shapes=[
            pltpu.VMEM((m, n), partial.dtype),
            pltpu.VMEM((m // 2, n), bf16),
            pltpu.VMEM((rs_total, n), bf16),
            pltpu.VMEM((m, n), bf16),
            pltpu.SemaphoreType.DMA((10,)),
            pltpu.SemaphoreType.DMA((10,)),
        ],
        compiler_params=pltpu.CompilerParams(collective_id=0),
    )(partial)


def _partial_attn(x, Wq, K, V, Wo):
    B, Sq, E = x.shape
    h_per = Wq.shape[1] // DH
    bf16 = jnp.bfloat16

    def body(x_ref, wq_ref, k_ref, v_ref, wo_ref, out_ref):
        qi = lax.broadcasted_iota(jnp.int32, (Sq, Sq), 0)
        ki = lax.broadcasted_iota(jnp.int32, (Sq, Sq), 1)
        neg = jnp.float32(-1e9)
        band = jnp.abs(qi - ki) <= WINDOW
        for b in range(B):
            xb = x_ref[b, :, :].astype(bf16)
            acc = jnp.zeros((Sq, E), jnp.float32)
            for h in range(h_per):
                wq_h = wq_ref[:, h * DH:(h + 1) * DH].astype(bf16)
                q = jnp.dot(xb, wq_h,
                            preferred_element_type=jnp.float32)
                q = (q * 0.125).astype(bf16)
                k = k_ref[b, :, h * DH:(h + 1) * DH].astype(bf16)
                s = lax.dot_general(
                    q, k, (((1,), (1,)), ((), ())),
                    preferred_element_type=jnp.float32)
                s = jnp.where(band, s, neg)
                s = s - jnp.max(s, axis=1, keepdims=True)
                e = jnp.exp(s)
                w = e / jnp.sum(e, axis=1, keepdims=True)
                v = v_ref[b, :, h * DH:(h + 1) * DH].astype(bf16)
                ctx = jnp.dot(w.astype(bf16), v,
                              preferred_element_type=jnp.float32)
                wo_h = wo_ref[h * DH:(h + 1) * DH, :].astype(bf16)
                acc = acc + jnp.dot(ctx.astype(bf16), wo_h,
                                    preferred_element_type=jnp.float32)
            out_ref[b * Sq:(b + 1) * Sq, :] = acc

    return pl.pallas_call(
        body,
        out_shape=jax.ShapeDtypeStruct((B * Sq, E), jnp.float32),
        in_specs=[pl.BlockSpec(memory_space=pltpu.VMEM)] * 5,
        out_specs=pl.BlockSpec(memory_space=pltpu.VMEM),
    )(x, Wq, K, V, Wo)


def kernel(x, Wq, K_ext, V_ext, Wo):
    my = lax.axis_index("i")
    B, Sq, E = x.shape
    h_per = Wq.shape[1] // DH

    K = lax.dynamic_slice_in_dim(K_ext, my * h_per, h_per, axis=2)
    V = lax.dynamic_slice_in_dim(V_ext, my * h_per, h_per, axis=2)
    K = K.reshape(B, Sq, h_per * DH)
    V = V.reshape(B, Sq, h_per * DH)

    partial = _partial_attn(x, Wq, K, V, Wo)
    out = _all_reduce(partial)
    return out.reshape(B, Sq, E)


# device time: 161129 ns/iter; 1.7069x vs baseline; 1.1378x over previous
import jax
import jax.numpy as jnp
from jax import lax
from jax.experimental import pallas as pl
from jax.experimental.pallas import tpu as pltpu

N_DEV = 32
DH = 64
WINDOW = 128


def _all_reduce(partial):
    m, n = partial.shape
    assert m % N_DEV == 0
    bf16 = jnp.bfloat16

    rs_sizes = [m >> (k + 1) for k in range(5)]
    rs_offs = [sum(rs_sizes[:k]) for k in range(5)]
    rs_total = sum(rs_sizes)

    def body(x_ref, out_ref, acc_ref, send_stage, rs_recv, ag_buf,
             send_sems, recv_sems):
        i = lax.axis_index("i")
        z = i // 8
        p = lax.rem(i, 8)
        y = p // 2
        x = jnp.bitwise_xor(lax.rem(p, 2), lax.rem(y, 2))

        def ring_idx(xx, yy, zz):
            pp = 2 * yy + jnp.bitwise_xor(xx, lax.rem(yy, 2))
            return 8 * zz + pp

        partners = [
            ring_idx(jnp.bitwise_xor(x, 1), y, z),
            ring_idx(x, jnp.bitwise_xor(y, 1), z),
            ring_idx(x, y, jnp.bitwise_xor(z, 1)),
            ring_idx(x, jnp.bitwise_xor(y, 2), z),
            ring_idx(x, y, jnp.bitwise_xor(z, 2)),
        ]
        bits = [
            lax.rem(x, 2),
            lax.rem(y, 2),
            lax.rem(z, 2),
            lax.rem(y // 2, 2),
            lax.rem(z // 2, 2),
        ]

        barrier_sem = pltpu.get_barrier_semaphore()
        for pk in partners:
            pl.semaphore_signal(
                barrier_sem, inc=1,
                device_id=(pk,), device_id_type=pl.DeviceIdType.MESH,
            )
        pl.semaphore_wait(barrier_sem, len(partners))

        acc_ref[:, :] = x_ref[:, :]

        seg_start = i * 0
        for k in range(5):
            half = rs_sizes[k]
            b = bits[k]
            send_off = seg_start + (1 - b) * half
            keep_off = seg_start + b * half
            send_stage[pl.ds(0, half), :] = acc_ref[
                pl.ds(send_off, half), :
            ].astype(bf16)
            rdma = pltpu.make_async_remote_copy(
                src_ref=send_stage.at[pl.ds(0, half)],
                dst_ref=rs_recv.at[pl.ds(rs_offs[k], half)],
                send_sem=send_sems.at[k],
                recv_sem=recv_sems.at[k],
                device_id=(partners[k],),
                device_id_type=pl.DeviceIdType.MESH,
            )
            rdma.start()
            rdma.wait()
            acc_ref[pl.ds(keep_off, half), :] = (
                acc_ref[pl.ds(keep_off, half), :]
                + rs_recv[pl.ds(rs_offs[k], half), :].astype(jnp.float32)
            )
            seg_start = keep_off

        own = m // N_DEV
        ag_buf[pl.ds(seg_start, own), :] = acc_ref[
            pl.ds(seg_start, own), :
        ].astype(bf16)
        own_start = seg_start
        for j in range(5):
            r = 4 - j
            b = bits[r]
            rdma = pltpu.make_async_remote_copy(
                src_ref=ag_buf.at[pl.ds(own_start, own)],
                dst_ref=ag_buf.at[pl.ds(own_start, own)],
                send_sem=send_sems.at[5 + j],
                recv_sem=recv_sems.at[5 + j],
                device_id=(partners[r],),
                device_id_type=pl.DeviceIdType.MESH,
            )
            rdma.start()
            rdma.wait()
            own_start = own_start - b * own
            own = own * 2

        out_ref[:, :] = ag_buf[:, :].astype(jnp.float32)

    return pl.pallas_call(
        body,
        out_shape=jax.ShapeDtypeStruct((m, n), partial.dtype),
        in_specs=[pl.BlockSpec(memory_space=pltpu.VMEM)],
        out_specs=pl.BlockSpec(memory_space=pltpu.VMEM),
        scratch_shapes=[
            pltpu.VMEM((m, n), partial.dtype),
            pltpu.VMEM((m // 2, n), bf16),
            pltpu.VMEM((rs_total, n), bf16),
            pltpu.VMEM((m, n), bf16),
            pltpu.SemaphoreType.DMA((10,)),
            pltpu.SemaphoreType.DMA((10,)),
        ],
        compiler_params=pltpu.CompilerParams(collective_id=0),
    )(partial)


def _partial_attn(x, Wq, K, V, Wo):
    B, Sq, E = x.shape
    h_per = Wq.shape[1] // DH
    bf16 = jnp.bfloat16

    def body(x_ref, wq_ref, k_ref, v_ref, wo_ref, out_ref):
        qi = lax.broadcasted_iota(jnp.int32, (Sq, Sq), 0)
        ki = lax.broadcasted_iota(jnp.int32, (Sq, Sq), 1)
        neg = jnp.float32(-1e9)
        band = jnp.abs(qi - ki) <= WINDOW
        for b in range(B):
            xb = x_ref[b, :, :].astype(bf16)
            acc = jnp.zeros((Sq, E), jnp.float32)
            for h in range(h_per):
                wq_h = wq_ref[:, h * DH:(h + 1) * DH].astype(bf16)
                q = jnp.dot(xb, wq_h,
                            preferred_element_type=jnp.float32)
                q = (q * 0.125).astype(bf16)
                k = k_ref[b, :, h * DH:(h + 1) * DH]
                s = lax.dot_general(
                    q, k, (((1,), (1,)), ((), ())),
                    preferred_element_type=jnp.float32)
                s = jnp.where(band, s, neg)
                s = s - jnp.max(s, axis=1, keepdims=True)
                e = jnp.exp(s)
                w = e / jnp.sum(e, axis=1, keepdims=True)
                v = v_ref[b, :, h * DH:(h + 1) * DH]
                ctx = jnp.dot(w.astype(bf16), v,
                              preferred_element_type=jnp.float32)
                wo_h = wo_ref[h * DH:(h + 1) * DH, :].astype(bf16)
                acc = acc + jnp.dot(ctx.astype(bf16), wo_h,
                                    preferred_element_type=jnp.float32)
            out_ref[b * Sq:(b + 1) * Sq, :] = acc

    return pl.pallas_call(
        body,
        out_shape=jax.ShapeDtypeStruct((B * Sq, E), jnp.float32),
        in_specs=[pl.BlockSpec(memory_space=pltpu.VMEM)] * 5,
        out_specs=pl.BlockSpec(memory_space=pltpu.VMEM),
    )(x, Wq, K, V, Wo)


def kernel(x, Wq, K_ext, V_ext, Wo):
    my = lax.axis_index("i")
    B, Sq, E = x.shape
    h_per = Wq.shape[1] // DH

    K = lax.dynamic_slice_in_dim(K_ext, my * h_per, h_per, axis=2)
    V = lax.dynamic_slice_in_dim(V_ext, my * h_per, h_per, axis=2)
    K = K.astype(jnp.bfloat16).reshape(B, Sq, h_per * DH)
    V = V.astype(jnp.bfloat16).reshape(B, Sq, h_per * DH)

    partial = _partial_attn(x, Wq, K, V, Wo)
    out = _all_reduce(partial)
    return out.reshape(B, Sq, E)
